# baseline (speedup 1.0000x reference)
import dataclasses
import functools

import jax
import jax.numpy as jnp
import numpy as np
from jax.experimental import pallas as pl
from jax.experimental.pallas import tpu as pltpu
from jax.experimental.pallas import tpu_sc as plsc

_L = 16


@functools.cache
def _make_lookup(B0, B1, N, V, D, dtype, scale):
    mesh = plsc.VectorSubcoreMesh(core_axis_name="core", subcore_axis_name="subcore")
    cp = pltpu.CompilerParams(use_tc_tiling_on_sc=False)
    if "needs_layout_passes" in pltpu.CompilerParams.__dataclass_fields__:
        cp = dataclasses.replace(cp, needs_layout_passes=False)

    n_full = N // _L
    tail = N % _L

    @functools.partial(
        pl.kernel,
        out_type=jax.ShapeDtypeStruct((B0, B1, N, D), dtype),
        mesh=mesh,
        scratch_types=[pltpu.VMEM((V * D,), dtype)],
        compiler_params=cp,
    )
    def lookup(table_hbm, idx_hbm, out_hbm, tab_vmem):
        pltpu.sync_copy(table_hbm, tab_vmem)

        @pl.loop(0, V * D // _L)
        def _(i):
            sl = pl.ds(i * _L, _L)
            tab_vmem[sl] = tab_vmem[sl] * scale

        iotas = [jax.lax.iota(jnp.int32, _L) + h * _L for h in range(D // _L)]

        def rows16(i_vmem, o_vmem, row0):
            idxv = i_vmem[0, 0, pl.ds(row0, _L)]
            addrs = idxv * D
            for r in range(_L):
                sel = jnp.full((_L,), r, jnp.int32)
                base = addrs.at[sel].get(mode="promise_in_bounds")
                for h, io in enumerate(iotas):
                    vals = plsc.load_gather(tab_vmem, [base + io])
                    o_vmem[0, 0, row0 + r, pl.ds(h * _L, _L)] = vals

        def body(i_vmem, o_vmem):
            @plsc.parallel_loop(0, n_full, unroll=2)
            def _(g):
                rows16(i_vmem, o_vmem, g * _L)

            if tail:
                rows16(i_vmem, o_vmem, N - _L)

        pltpu.emit_pipeline(
            body,
            grid=(B0 * B1,),
            in_specs=[pl.BlockSpec((1, 1, N), index_map=lambda i: (i, 0, 0))],
            out_specs=[
                pl.BlockSpec(
                    (1, 1, N, D), index_map=lambda i: (i // B1, i % B1, 0, 0)
                )
            ],
            core_axis_name=("core", "subcore"),
            dimension_semantics=(pltpu.PARALLEL,),
        )(idx_hbm, out_hbm)

    return lookup


def kernel(x, table):
    V, D = table.shape
    B0, B1, N = x.shape
    scale = float(np.sqrt(D).astype(np.float32))
    idx = x.reshape(B0 * B1, 1, N)
    return _make_lookup(B0, B1, N, V, D, table.dtype, scale)(
        table.reshape(V * D), idx
    )


# speedup vs baseline: 5.7845x; 5.7845x over previous
import dataclasses
import functools

import jax
import jax.numpy as jnp
import numpy as np
from jax.experimental import pallas as pl
from jax.experimental.pallas import tpu as pltpu
from jax.experimental.pallas import tpu_sc as plsc

_L = 16


@functools.cache
def _make_lookup(B0, B1, N, V, D, dtype, scale):
    mesh = plsc.VectorSubcoreMesh(core_axis_name="core", subcore_axis_name="subcore")
    cp = pltpu.CompilerParams(use_tc_tiling_on_sc=True)
    if "needs_layout_passes" in pltpu.CompilerParams.__dataclass_fields__:
        cp = dataclasses.replace(cp, needs_layout_passes=False)

    P = D + 1
    n_full = N // _L
    tail = N % _L

    @functools.partial(
        pl.kernel,
        out_type=jax.ShapeDtypeStruct((B0, B1, D, N), dtype),
        mesh=mesh,
        scratch_types=[pltpu.VMEM((V * P,), dtype)],
        compiler_params=cp,
    )
    def lookup(table_hbm, idx_hbm, out_hbm, tab_vmem):
        pltpu.sync_copy(table_hbm, tab_vmem.at[pl.ds(V, V * D)])

        @pl.loop(0, V)
        def _(r):
            for h in range(D // _L):
                src = pl.ds(V + r * D + h * _L, _L)
                tab_vmem[pl.ds(r * P + h * _L, _L)] = tab_vmem[src] * scale

        def grp(i_vmem, o_vmem, v0):
            idxv = i_vmem[0, 0, pl.ds(v0, _L)]
            addrs = idxv * P
            for d in range(D):
                vals = plsc.load_gather(tab_vmem, [addrs + d])
                o_vmem[0, 0, d, pl.ds(v0, _L)] = vals

        def body(i_vmem, o_vmem):
            @plsc.parallel_loop(0, n_full, unroll=2)
            def _(g):
                grp(i_vmem, o_vmem, g * _L)

            if tail:
                grp(i_vmem, o_vmem, N - _L)

        pltpu.emit_pipeline(
            body,
            grid=(B0 * B1,),
            in_specs=[pl.BlockSpec((1, 1, N), index_map=lambda i: (i, 0, 0))],
            out_specs=[
                pl.BlockSpec(
                    (1, 1, D, N), index_map=lambda i: (i // B1, i % B1, 0, 0)
                )
            ],
            core_axis_name=("core", "subcore"),
            dimension_semantics=(pltpu.PARALLEL,),
        )(idx_hbm, out_hbm)

    return lookup


def kernel(x, table):
    V, D = table.shape
    B0, B1, N = x.shape
    scale = float(np.sqrt(D).astype(np.float32))
    idx = x.reshape(B0 * B1, 1, N)
    out = _make_lookup(B0, B1, N, V, D, table.dtype, scale)(
        table.reshape(V * D), idx
    )
    return jnp.swapaxes(out, 2, 3)
